# baseline (speedup 1.0000x reference)
import jax
import jax.numpy as jnp
from jax import lax
from jax.experimental import pallas as pl
from jax.experimental.pallas import tpu as pltpu
from jax.experimental.pallas import tpu_sc as plsc

N_CODEBOOKS = 8
VOCAB = 2048
DIM = 512
B = 8
T = 2048

_INFO = plsc.get_sparse_core_info()
_NC, _NS, _L = _INFO.num_cores, _INFO.num_subcores, _INFO.num_lanes
_NW = _NC * _NS
_ROWS = B * N_CODEBOOKS * T
_RW = _ROWS // _NW
_PAIRS_PER_W = (B * N_CODEBOOKS) // _NW
_CHUNK = 32
_NCHUNK = _RW // _CHUNK
_NBUF = 6
_GF = 3
_OF = 3
_MAIN = (_NCHUNK // _NBUF) * _NBUF


def _sc_body(x_hbm, tab_hbm, out_hbm, idx_v, b0, b1, b2, b3, b4, b5,
             gsem, osem):
    wid = lax.axis_index("s") * _NC + lax.axis_index("c")
    bufs = (b0, b1, b2, b3, b4, b5)
    base = wid * _RW

    pltpu.sync_copy(x_hbm.at[pl.ds(base, _RW)], idx_v)
    for j in range(_PAIRS_PER_W):
        pair = wid * _PAIRS_PER_W + j
        offset = lax.rem(pair, N_CODEBOOKS) * VOCAB

        def _add_off(i, _):
            sl = pl.ds(j * T + i * _L, _L)
            idx_v[sl] = idx_v[sl] + offset
            return 0

        lax.fori_loop(0, T // _L, _add_off, 0, unroll=8)

    def _gather(c, buf):
        pltpu.async_copy(tab_hbm.at[idx_v.at[pl.ds(c * _CHUNK, _CHUNK)]],
                         buf, gsem)

    def _wait_gather(buf):
        pltpu.make_async_copy(
            tab_hbm.at[idx_v.at[pl.ds(0, _CHUNK)]], buf, gsem).wait()

    def _out(c, buf):
        pltpu.async_copy(buf, out_hbm.at[pl.ds(base + c * _CHUNK, _CHUNK)],
                         osem)

    def _wait_out(buf):
        pltpu.make_async_copy(
            buf, out_hbm.at[pl.ds(base, _CHUNK)], osem).wait()

    for c in range(_GF):
        _gather(c, bufs[c])

    def _step(c, p, g_pred, o_pred):
        _wait_gather(bufs[p])

        if o_pred:
            @pl.when(c >= _OF)
            def _():
                _wait_out(bufs[(p - _OF) % _NBUF])
        elif c >= _OF:
            _wait_out(bufs[(p - _OF) % _NBUF])

        if g_pred:
            @pl.when(c + _GF < _NCHUNK)
            def _():
                _gather(c + _GF, bufs[(p - _OF) % _NBUF])
        elif c + _GF < _NCHUNK:
            _gather(c + _GF, bufs[(p - _OF) % _NBUF])

        _out(c, bufs[p])

    def _chunks(cn, _):
        for p in range(_NBUF):
            c = cn * _NBUF + p
            _step(c, p, True, True)
        return 0

    lax.fori_loop(0, _MAIN // _NBUF, _chunks, 0)

    for c in range(_MAIN, _NCHUNK):
        _step(c, c % _NBUF, False, False)

    for c in range(_NCHUNK - _OF, _NCHUNK):
        _wait_out(bufs[c % _NBUF])


@jax.jit
def _embed(x_flat, tab_flat):
    mesh = plsc.VectorSubcoreMesh(core_axis_name="c", subcore_axis_name="s")
    return pl.kernel(
        _sc_body,
        out_type=jax.ShapeDtypeStruct((_ROWS, DIM), jnp.float32),
        mesh=mesh,
        scratch_types=[
            pltpu.VMEM((_RW,), jnp.int32),
            pltpu.VMEM((_CHUNK, DIM), jnp.float32),
            pltpu.VMEM((_CHUNK, DIM), jnp.float32),
            pltpu.VMEM((_CHUNK, DIM), jnp.float32),
            pltpu.VMEM((_CHUNK, DIM), jnp.float32),
            pltpu.VMEM((_CHUNK, DIM), jnp.float32),
            pltpu.VMEM((_CHUNK, DIM), jnp.float32),
            pltpu.SemaphoreType.DMA,
            pltpu.SemaphoreType.DMA,
        ],
    )(x_flat, tab_flat)


def kernel(x, tables):
    x_flat = x.reshape(_ROWS)
    tab_flat = tables.reshape(N_CODEBOOKS * VOCAB, DIM)
    out = _embed(x_flat, tab_flat)
    return out.reshape(B, N_CODEBOOKS, T, DIM)


# speedup vs baseline: 1.0031x; 1.0031x over previous
import jax
import jax.numpy as jnp
from jax import lax
from jax.experimental import pallas as pl
from jax.experimental.pallas import tpu as pltpu
from jax.experimental.pallas import tpu_sc as plsc

N_CODEBOOKS = 8
VOCAB = 2048
DIM = 512
B = 8
T = 2048

_INFO = plsc.get_sparse_core_info()
_NC, _NS, _L = _INFO.num_cores, _INFO.num_subcores, _INFO.num_lanes
_NW = _NC * _NS
_ROWS = B * N_CODEBOOKS * T
_RW = _ROWS // _NW
_PAIRS_PER_W = (B * N_CODEBOOKS) // _NW
_CHUNK = 32
_NCHUNK = _RW // _CHUNK
_NBUF = 6
_GF = 3
_OF = 3
_MAIN = (_NCHUNK // _NBUF) * _NBUF


def _sc_body(x_hbm, tab_hbm, out_hbm, idx_v, b0, b1, b2, b3, b4, b5,
             gsem, osem):
    wid = lax.axis_index("s") * _NC + lax.axis_index("c")
    bufs = (b0, b1, b2, b3, b4, b5)
    base = wid * _RW

    pltpu.sync_copy(x_hbm.at[pl.ds(base, _RW)], idx_v)

    _CPP = T // _CHUNK

    def _gather(c, buf):
        k = lax.rem(wid * _PAIRS_PER_W + c // _CPP, N_CODEBOOKS)
        koff = pl.multiple_of(k * VOCAB, VOCAB)
        pltpu.async_copy(
            tab_hbm.at[pl.ds(koff, VOCAB)].at[
                idx_v.at[pl.ds(c * _CHUNK, _CHUNK)]],
            buf, gsem)

    def _wait_gather(buf):
        pltpu.make_async_copy(
            tab_hbm.at[idx_v.at[pl.ds(0, _CHUNK)]], buf, gsem).wait()

    def _out(c, buf):
        pltpu.async_copy(buf, out_hbm.at[pl.ds(base + c * _CHUNK, _CHUNK)],
                         osem)

    def _wait_out(buf):
        pltpu.make_async_copy(
            buf, out_hbm.at[pl.ds(base, _CHUNK)], osem).wait()

    for c in range(_GF):
        _gather(c, bufs[c])

    def _step(c, p, g_pred, o_pred):
        _wait_gather(bufs[p])

        if o_pred:
            @pl.when(c >= _OF)
            def _():
                _wait_out(bufs[(p - _OF) % _NBUF])
        elif c >= _OF:
            _wait_out(bufs[(p - _OF) % _NBUF])

        if g_pred:
            @pl.when(c + _GF < _NCHUNK)
            def _():
                _gather(c + _GF, bufs[(p - _OF) % _NBUF])
        elif c + _GF < _NCHUNK:
            _gather(c + _GF, bufs[(p - _OF) % _NBUF])

        _out(c, bufs[p])

    def _chunks(cn, _):
        for p in range(_NBUF):
            c = cn * _NBUF + p
            _step(c, p, True, True)
        return 0

    lax.fori_loop(0, _MAIN // _NBUF, _chunks, 0)

    for c in range(_MAIN, _NCHUNK):
        _step(c, c % _NBUF, False, False)

    for c in range(_NCHUNK - _OF, _NCHUNK):
        _wait_out(bufs[c % _NBUF])


@jax.jit
def _embed(x_flat, tab_flat):
    mesh = plsc.VectorSubcoreMesh(core_axis_name="c", subcore_axis_name="s")
    return pl.kernel(
        _sc_body,
        out_type=jax.ShapeDtypeStruct((_ROWS, DIM), jnp.float32),
        mesh=mesh,
        scratch_types=[
            pltpu.VMEM((_RW,), jnp.int32),
            pltpu.VMEM((_CHUNK, DIM), jnp.float32),
            pltpu.VMEM((_CHUNK, DIM), jnp.float32),
            pltpu.VMEM((_CHUNK, DIM), jnp.float32),
            pltpu.VMEM((_CHUNK, DIM), jnp.float32),
            pltpu.VMEM((_CHUNK, DIM), jnp.float32),
            pltpu.VMEM((_CHUNK, DIM), jnp.float32),
            pltpu.SemaphoreType.DMA,
            pltpu.SemaphoreType.DMA,
        ],
    )(x_flat, tab_flat)


def kernel(x, tables):
    x_flat = x.reshape(_ROWS)
    tab_flat = tables.reshape(N_CODEBOOKS * VOCAB, DIM)
    out = _embed(x_flat, tab_flat)
    return out.reshape(B, N_CODEBOOKS, T, DIM)
